# baseline (speedup 1.0000x reference)
import jax
import jax.numpy as jnp
from jax import lax
from jax.experimental import pallas as pl
from jax.experimental.pallas import tpu as pltpu
from jax.experimental.pallas import tpu_sc as plsc

BATCH = 16384
EMBED = 16
_NC = 2
_NS = 16
_NW = _NC * _NS
_BPW = BATCH // _NW
_NCHUNK = 4
_CHUNK = _BPW // _NCHUNK
_UNROLL = 8


def _sc_body(uidx_hbm, bidx_hbm, uemb_hbm, ubias_hbm, bemb_hbm, bbias_hbm,
             partials_hbm, ubb_hbm,
             uidx_v, bidx_v, urows_v, brows_v, ub_v, bb_v, acc_v,
             sem_rows, sem_bias):
    c = lax.axis_index("c")
    s = lax.axis_index("s")
    wid = c * _NS + s
    base = wid * _BPW

    pltpu.sync_copy(uidx_hbm.at[wid], uidx_v)
    pltpu.sync_copy(bidx_hbm.at[wid], bidx_v)

    row_copies = []
    bias_copies = []
    for j in range(_NCHUNK):
        dst = pl.ds(j * _CHUNK, _CHUNK)
        row_copies.append(
            pltpu.async_copy(uemb_hbm.at[uidx_v.at[j]], urows_v.at[dst], sem_rows))
        row_copies.append(
            pltpu.async_copy(bemb_hbm.at[bidx_v.at[j]], brows_v.at[dst], sem_rows))
        bias_copies.append(
            pltpu.async_copy(ubias_hbm.at[uidx_v.at[j]], ub_v.at[dst], sem_bias))
        bias_copies.append(
            pltpu.async_copy(bbias_hbm.at[bidx_v.at[j]], bb_v.at[dst], sem_bias))
    for cp in row_copies:
        cp.wait()

    zero = jnp.zeros((EMBED,), jnp.float32)

    def dot_step(i, accs):
        r = i * _UNROLL
        accs = list(accs)
        for j in range(_UNROLL):
            accs[j % 4] = accs[j % 4] + urows_v[r + j, :] * brows_v[r + j, :]
        return tuple(accs)

    a0, a1, a2, a3 = lax.fori_loop(
        0, _BPW // _UNROLL, dot_step, (zero, zero, zero, zero))
    acc_v[...] = (a0 + a1) + (a2 + a3)
    pltpu.sync_copy(acc_v, partials_hbm.at[wid])

    for cp in bias_copies:
        cp.wait()
    for i in range(_BPW // EMBED):
        off = pl.ds(i * EMBED, EMBED)
        ub_v[off] = ub_v[off] + bb_v[off]
    pltpu.sync_copy(ub_v, ubb_hbm.at[pl.ds(base, _BPW)])


_gather_dot = pl.kernel(
    _sc_body,
    out_type=(
        jax.ShapeDtypeStruct((_NW, EMBED), jnp.float32),
        jax.ShapeDtypeStruct((BATCH,), jnp.float32),
    ),
    mesh=plsc.VectorSubcoreMesh(core_axis_name="c", subcore_axis_name="s"),
    scratch_types=(
        pltpu.VMEM((_NCHUNK, _CHUNK), jnp.int32),
        pltpu.VMEM((_NCHUNK, _CHUNK), jnp.int32),
        pltpu.VMEM((_BPW, EMBED), jnp.float32),
        pltpu.VMEM((_BPW, EMBED), jnp.float32),
        pltpu.VMEM((_BPW,), jnp.float32),
        pltpu.VMEM((_BPW,), jnp.float32),
        pltpu.VMEM((EMBED,), jnp.float32),
        pltpu.SemaphoreType.DMA,
        pltpu.SemaphoreType.DMA,
    ),
    compiler_params=pltpu.CompilerParams(use_tc_tiling_on_sc=False),
)


def _tc_body(partials_ref, ubb_ref, out_ref):
    s = jnp.sum(partials_ref[...])
    x = ubb_ref[...] + s
    out_ref[...] = 1.0 / (1.0 + jnp.exp(-x))


_finish = pl.pallas_call(
    _tc_body,
    out_shape=jax.ShapeDtypeStruct((128, 128), jnp.float32),
)


def kernel(inputs, user_embedding, user_bias, business_embedding, business_bias):
    uidx = inputs[:, 0].reshape(_NW, _NCHUNK, _CHUNK)
    bidx = inputs[:, 1].reshape(_NW, _NCHUNK, _CHUNK)
    ubias = user_bias.reshape(-1)
    bbias = business_bias.reshape(-1)
    partials, ubb = _gather_dot(
        uidx, bidx, user_embedding, ubias, business_embedding, bbias)
    out = _finish(partials, ubb.reshape(128, 128))
    return out.reshape(BATCH, 1)
